# Initial kernel scaffold; baseline (speedup 1.0000x reference)
import functools

import jax
import jax.numpy as jnp
from jax import lax
from jax.experimental import pallas as pl
from jax.experimental.pallas import tpu as pltpu
from jax.experimental.pallas import tpu_sc as plsc

N = 1000000
D = 64
B = 16384
K = 20
POS_MARGIN = 2.0
NEG_MARGIN = 1.0
ZERO_MARGIN = 0.5

NC = 2
NS = 16
NW = NC * NS

CH = 512


def _sc_gather(H, T, h, t, nt_flat):
    BK = nt_flat.shape[0]
    bw = B // NW
    nw = BK // NW
    mesh = plsc.VectorSubcoreMesh(
        core_axis_name="c", subcore_axis_name="s", num_cores=NC, num_subcores=NS
    )

    @functools.partial(
        pl.kernel,
        out_type=(
            jax.ShapeDtypeStruct((B, D), jnp.float32),
            jax.ShapeDtypeStruct((B, D), jnp.float32),
            jax.ShapeDtypeStruct((BK, D), jnp.float32),
        ),
        mesh=mesh,
        scratch_types=[
            pltpu.VMEM((CH,), jnp.int32),
            pltpu.VMEM((CH, D), jnp.float32),
            pltpu.SemaphoreType.DMA,
        ],
    )
    def k(H_hbm, T_hbm, h_hbm, t_hbm, nt_hbm, hr_hbm, tr_hbm, ntr_hbm,
          idx_v, rows_v, sem):
        wid = lax.axis_index("s") * NC + lax.axis_index("c")
        base = wid * bw
        pltpu.sync_copy(h_hbm.at[pl.ds(base, bw)], idx_v)
        pltpu.async_copy(H_hbm.at[idx_v], rows_v, sem).wait()
        pltpu.sync_copy(rows_v, hr_hbm.at[pl.ds(base, bw)])
        pltpu.sync_copy(t_hbm.at[pl.ds(base, bw)], idx_v)
        pltpu.async_copy(T_hbm.at[idx_v], rows_v, sem).wait()
        pltpu.sync_copy(rows_v, tr_hbm.at[pl.ds(base, bw)])

        nbase = wid * nw

        @pl.loop(0, nw, step=CH)
        def _(off):
            pltpu.sync_copy(nt_hbm.at[pl.ds(nbase + off, CH)], idx_v)
            pltpu.async_copy(T_hbm.at[idx_v], rows_v, sem).wait()
            pltpu.sync_copy(rows_v, ntr_hbm.at[pl.ds(nbase + off, CH)])

    return k(H, T, h, t, nt_flat)


BB = 2048


def _tc_loss_kernel(h_ref, t_ref, nt_ref, s_ref, r_ref, nr_ref, R_ref, out_ref):
    hv = h_ref[...]
    tv = t_ref[...]
    sv = s_ref[...]
    ri = r_ref[...]
    R0 = R_ref[0:1, :]
    R1 = R_ref[1:2, :]
    R2 = R_ref[2:3, :]
    r_emb = jnp.where(ri == 0, R0, jnp.where(ri == 1, R1, R2))
    pos = jnp.sum(jnp.abs(hv + sv * r_emb - tv), axis=1, keepdims=True)
    acc = jnp.float32(0.0)
    for k in range(K):
        ntk = nt_ref[k]
        nrk = nr_ref[k]
        rk = jnp.where(nrk == 0, R0, jnp.where(nrk == 1, R1, R2))
        neg = jnp.sum(jnp.abs(hv + sv * rk - ntk), axis=1, keepdims=True)
        m = jnp.where(nrk == 1, POS_MARGIN,
                      jnp.where(nrk == 0, NEG_MARGIN, ZERO_MARGIN))
        acc += jnp.sum(jnp.maximum(0.0, m + pos - neg))

    @pl.when(pl.program_id(0) == 0)
    def _():
        out_ref[0, 0] = 0.0

    out_ref[0, 0] += acc


def _tc_loss(hrows, trows, ntrows_kbd, sign_f, r_i, nr_kbd, R_pad):
    grid = (B // BB,)
    return pl.pallas_call(
        _tc_loss_kernel,
        grid=grid,
        in_specs=[
            pl.BlockSpec((BB, D), lambda i: (i, 0)),
            pl.BlockSpec((BB, D), lambda i: (i, 0)),
            pl.BlockSpec((K, BB, D), lambda i: (0, i, 0)),
            pl.BlockSpec((BB, 1), lambda i: (i, 0)),
            pl.BlockSpec((BB, 1), lambda i: (i, 0)),
            pl.BlockSpec((K, BB, 1), lambda i: (0, i, 0)),
            pl.BlockSpec((8, D), lambda i: (0, 0)),
        ],
        out_specs=pl.BlockSpec((1, 1), lambda i: (0, 0)),
        out_shape=jax.ShapeDtypeStruct((1, 1), jnp.float32),
    )(hrows, trows, ntrows_kbd, sign_f, r_i, nr_kbd, R_pad)


def kernel(h, r, t, sign, negs_r, negs_t, H, R, T):
    h = h.astype(jnp.int32)
    t = t.astype(jnp.int32)
    nt_flat = negs_t.astype(jnp.int32).T.reshape(B * K)
    hrows, trows, ntrows = _sc_gather(H, T, h, t, nt_flat)
    ntrows_kbd = ntrows.reshape(K, B, D)
    sign_f = sign.astype(jnp.float32).reshape(B, 1)
    r_i = r.astype(jnp.int32).reshape(B, 1)
    nr_kbd = negs_r.astype(jnp.int32).T.reshape(K, B, 1)
    R_pad = jnp.zeros((8, D), jnp.float32).at[:3].set(R)
    out = _tc_loss(hrows, trows, ntrows_kbd, sign_f, r_i, nr_kbd, R_pad)
    return out.reshape(())


# speedup vs baseline: 1.3873x; 1.3873x over previous
import functools

import jax
import jax.numpy as jnp
from jax import lax
from jax.experimental import pallas as pl
from jax.experimental.pallas import tpu as pltpu
from jax.experimental.pallas import tpu_sc as plsc

N = 1000000
D = 64
B = 16384
K = 20
POS_MARGIN = 2.0
NEG_MARGIN = 1.0
ZERO_MARGIN = 0.5

NC = 2
NS = 16
NW = NC * NS

CH = 512


def _sc_gather(H, T, h, t, nt_flat):
    BK = nt_flat.shape[0]
    bw = B // NW
    nw = BK // NW
    mesh = plsc.VectorSubcoreMesh(
        core_axis_name="c", subcore_axis_name="s", num_cores=NC, num_subcores=NS
    )

    @functools.partial(
        pl.kernel,
        out_type=(
            jax.ShapeDtypeStruct((B, D), jnp.float32),
            jax.ShapeDtypeStruct((B, D), jnp.float32),
            jax.ShapeDtypeStruct((BK, D), jnp.float32),
        ),
        mesh=mesh,
        scratch_types=[
            pltpu.VMEM((CH,), jnp.int32),
            pltpu.VMEM((CH, D), jnp.float32),
            pltpu.SemaphoreType.DMA,
        ],
        compiler_params=pltpu.CompilerParams(use_tc_tiling_on_sc=False),
    )
    def k(H_hbm, T_hbm, h_hbm, t_hbm, nt_hbm, hr_hbm, tr_hbm, ntr_hbm,
          idx_v, rows_v, sem):
        wid = lax.axis_index("s") * NC + lax.axis_index("c")
        base = wid * bw
        pltpu.sync_copy(h_hbm.at[pl.ds(base, bw)], idx_v)
        pltpu.async_copy(H_hbm.at[idx_v], rows_v, sem).wait()
        pltpu.sync_copy(rows_v, hr_hbm.at[pl.ds(base, bw)])
        pltpu.sync_copy(t_hbm.at[pl.ds(base, bw)], idx_v)
        pltpu.async_copy(T_hbm.at[idx_v], rows_v, sem).wait()
        pltpu.sync_copy(rows_v, tr_hbm.at[pl.ds(base, bw)])

        nbase = wid * nw

        @pl.loop(0, nw, step=CH)
        def _(off):
            pltpu.sync_copy(nt_hbm.at[pl.ds(nbase + off, CH)], idx_v)
            pltpu.async_copy(T_hbm.at[idx_v], rows_v, sem).wait()
            pltpu.sync_copy(rows_v, ntr_hbm.at[pl.ds(nbase + off, CH)])

    return k(H, T, h, t, nt_flat)


BB = 512


def _tc_loss_kernel(h_ref, t_ref, nt_ref, s_ref, r_ref, nr_ref, R_ref, out_ref):
    hv = h_ref[...]
    tv = t_ref[...]
    sv = s_ref[...]
    ri = r_ref[...]
    R0 = R_ref[0:1, :]
    R1 = R_ref[1:2, :]
    R2 = R_ref[2:3, :]
    r_emb = jnp.where(ri == 0, R0, jnp.where(ri == 1, R1, R2))
    pos = jnp.sum(jnp.abs(hv + sv * r_emb - tv), axis=1, keepdims=True)
    acc = jnp.float32(0.0)
    for k in range(K):
        ntk = nt_ref[k]
        nrk = nr_ref[k]
        rk = jnp.where(nrk == 0, R0, jnp.where(nrk == 1, R1, R2))
        neg = jnp.sum(jnp.abs(hv + sv * rk - ntk), axis=1, keepdims=True)
        m = jnp.where(nrk == 1, POS_MARGIN,
                      jnp.where(nrk == 0, NEG_MARGIN, ZERO_MARGIN))
        acc += jnp.sum(jnp.maximum(0.0, m + pos - neg))

    @pl.when(pl.program_id(0) == 0)
    def _():
        out_ref[...] = jnp.zeros_like(out_ref)

    out_ref[...] = out_ref[...] + acc


def _tc_loss(hrows, trows, ntrows_kbd, sign_f, r_i, nr_kbd, R_pad):
    grid = (B // BB,)
    return pl.pallas_call(
        _tc_loss_kernel,
        grid=grid,
        in_specs=[
            pl.BlockSpec((BB, D), lambda i: (i, 0)),
            pl.BlockSpec((BB, D), lambda i: (i, 0)),
            pl.BlockSpec((K, BB, D), lambda i: (0, i, 0)),
            pl.BlockSpec((BB, 1), lambda i: (i, 0)),
            pl.BlockSpec((BB, 1), lambda i: (i, 0)),
            pl.BlockSpec((K, BB, 1), lambda i: (0, i, 0)),
            pl.BlockSpec((8, D), lambda i: (0, 0)),
        ],
        out_specs=pl.BlockSpec((1, 1), lambda i: (0, 0)),
        out_shape=jax.ShapeDtypeStruct((1, 1), jnp.float32),
    )(hrows, trows, ntrows_kbd, sign_f, r_i, nr_kbd, R_pad)


def kernel(h, r, t, sign, negs_r, negs_t, H, R, T):
    h = h.astype(jnp.int32)
    t = t.astype(jnp.int32)
    nt_flat = negs_t.astype(jnp.int32).T.reshape(B * K)
    hrows, trows, ntrows = _sc_gather(H, T, h, t, nt_flat)
    ntrows_kbd = ntrows.reshape(K, B, D)
    sign_f = sign.astype(jnp.float32).reshape(B, 1)
    r_i = r.astype(jnp.int32).reshape(B, 1)
    nr_kbd = negs_r.astype(jnp.int32).T.reshape(K, B, 1)
    R_pad = jnp.zeros((8, D), jnp.float32).at[:3].set(R)
    out = _tc_loss(hrows, trows, ntrows_kbd, sign_f, r_i, nr_kbd, R_pad)
    return out.reshape(())
